# baseline (speedup 1.0000x reference)
import functools

import jax
import jax.numpy as jnp
import numpy as np
from jax import lax
from jax.experimental import pallas as pl
from jax.experimental.pallas import tpu as pltpu
from jax.experimental.pallas import tpu_sc as plsc

F32 = jnp.float32
BF16 = jnp.bfloat16
U32 = jnp.uint32
D = 256
W_TAB = 256
NUM_WORKERS = 32
GATHER_CHUNK = 128
HI_MASK = np.uint32(0xFFFF0000)


def _pack_bf16(x0, x1):
    a0 = lax.bitcast_convert_type(x0.astype(BF16).astype(F32), U32)
    a1 = lax.bitcast_convert_type(x1.astype(BF16).astype(F32), U32)
    return lax.bitcast_convert_type((a0 >> 16) | (a1 & HI_MASK), jnp.int32)


def _unpack_lo(gi):
    u = lax.bitcast_convert_type(gi, U32)
    return lax.bitcast_convert_type(u << 16, F32)


def _unpack_hi(gi):
    u = lax.bitcast_convert_type(gi, U32)
    return lax.bitcast_convert_type(u & HI_MASK, F32)


def _knn_body(K, N, qx_ref, kxt_ref, out_ref):
    qx = qx_ref[0]
    kxt = kxt_ref[0]
    b = pl.program_id(0)
    d = (jnp.sum(qx * qx, axis=1, keepdims=True)
         + jnp.sum(kxt * kxt, axis=0, keepdims=True)
         - 2.0 * jnp.dot(qx, kxt, preferred_element_type=F32))
    H = N // 2
    lo = d[:, :H]
    hi = d[:, H:]
    key = jnp.minimum(lo, hi)
    alt = jnp.maximum(lo, hi)
    iota = lax.broadcasted_iota(jnp.int32, key.shape, 1)
    oidx = jnp.where(hi < lo, iota + H, iota)
    cols = []
    for _ in range(K):
        m = jnp.min(key, axis=1, keepdims=True)
        oi = jnp.min(jnp.where(key <= m, oidx, 2 * N), axis=1, keepdims=True)
        cols.append(oi)
        cl = oi & (H - 1)
        pidx = jnp.where(oi >= H, oi - H, oi + H)
        eqc = iota == cl
        key = jnp.where(eqc, alt, key)
        alt = jnp.where(eqc, jnp.inf, alt)
        oidx = jnp.where(eqc, pidx, oidx)
    out_ref[...] = jnp.concatenate(cols, axis=1) + b * N


def _knn(qx16, kxt16, K, BQ):
    B, NQ, _ = qx16.shape
    N = kxt16.shape[2]
    grid = (B, NQ // BQ)
    return pl.pallas_call(
        functools.partial(_knn_body, K, N),
        grid=grid,
        in_specs=[
            pl.BlockSpec((1, BQ, 16), lambda b, q: (b, q, 0)),
            pl.BlockSpec((1, 16, N), lambda b, q: (b, 0, 0)),
        ],
        out_specs=pl.BlockSpec((BQ, K), lambda b, q: (b * (NQ // BQ) + q, 0)),
        out_shape=jax.ShapeDtypeStruct((B * NQ, K), jnp.int32),
    )(qx16, kxt16)


def _proj_q_body(qf_ref, qx16_ref, fc1w, fc1b, wq, q_out, tab_out):
    x = jnp.dot(qf_ref[...], fc1w[...], preferred_element_type=F32) + fc1b[...]
    q_out[...] = jnp.dot(x, wq[...], preferred_element_type=F32)
    qx = qx16_ref[...]
    tab_out[:, 0:128] = _pack_bf16(x[:, 0:128], x[:, 128:256])
    tab_out[:, 128:144] = lax.bitcast_convert_type(qx, jnp.int32)


def _proj_k_body(kf_ref, kx16_ref, fc1w, fc1b, tab_out):
    x = jnp.dot(kf_ref[...], fc1w[...], preferred_element_type=F32) + fc1b[...]
    kx = kx16_ref[...]
    tab_out[:, 0:128] = _pack_bf16(x[:, 0:128], x[:, 128:256])
    tab_out[:, 128:144] = lax.bitcast_convert_type(kx, jnp.int32)


def _wspec(shape):
    return pl.BlockSpec(shape, lambda i: (0,) * len(shape))


def _proj_query(qf, qx16, fc1w, fc1b, wq, RP=512):
    R = qf.shape[0]
    grid = (R // RP,)
    return pl.pallas_call(
        _proj_q_body,
        grid=grid,
        in_specs=[
            pl.BlockSpec((RP, 256), lambda i: (i, 0)),
            pl.BlockSpec((RP, 16), lambda i: (i, 0)),
            _wspec((256, 256)), _wspec((1, 256)),
            _wspec((256, 256)),
        ],
        out_specs=(
            pl.BlockSpec((RP, 256), lambda i: (i, 0)),
            pl.BlockSpec((RP, W_TAB), lambda i: (i, 0)),
        ),
        out_shape=(
            jax.ShapeDtypeStruct((R, 256), F32),
            jax.ShapeDtypeStruct((R, W_TAB), jnp.int32),
        ),
    )(qf, qx16, fc1w, fc1b, wq)


def _proj_key(kf, kx16, fc1w, fc1b, RP=512):
    R = kf.shape[0]
    grid = (R // RP,)
    return pl.pallas_call(
        _proj_k_body,
        grid=grid,
        in_specs=[
            pl.BlockSpec((RP, 256), lambda i: (i, 0)),
            pl.BlockSpec((RP, 16), lambda i: (i, 0)),
            _wspec((256, 256)), _wspec((1, 256)),
        ],
        out_specs=pl.BlockSpec((RP, W_TAB), lambda i: (i, 0)),
        out_shape=jax.ShapeDtypeStruct((R, W_TAB), jnp.int32),
    )(kf, kx16, fc1w, fc1b)


def _make_gather(R, W, CH):
    mesh = plsc.VectorSubcoreMesh(core_axis_name="c", subcore_axis_name="s")
    per_w = R // NUM_WORKERS

    @functools.partial(
        pl.kernel, mesh=mesh,
        out_type=jax.ShapeDtypeStruct((R, W), jnp.int32),
        scratch_types=[
            pltpu.VMEM((CH,), jnp.int32),
            pltpu.VMEM((CH,), jnp.int32),
            pltpu.VMEM((CH, W), jnp.int32),
            pltpu.VMEM((CH, W), jnp.int32),
            pltpu.SemaphoreType.DMA,
            pltpu.SemaphoreType.DMA,
        ],
    )
    def gk(tab_hbm, idx_hbm, out_hbm, idx_v0, idx_v1, rows_v0, rows_v1,
           sem0, sem1):
        wid = lax.axis_index("s") * 2 + lax.axis_index("c")
        base = wid * per_w

        def body(j, carry):
            b0 = base + (2 * j) * CH
            b1 = b0 + CH
            pltpu.sync_copy(idx_hbm.at[pl.ds(b0, CH)], idx_v0)
            g0 = pltpu.async_copy(tab_hbm.at[idx_v0], rows_v0, sem0)
            pltpu.sync_copy(idx_hbm.at[pl.ds(b1, CH)], idx_v1)
            g1 = pltpu.async_copy(tab_hbm.at[idx_v1], rows_v1, sem1)
            g0.wait()
            pltpu.sync_copy(rows_v0, out_hbm.at[pl.ds(b0, CH)])
            g1.wait()
            pltpu.sync_copy(rows_v1, out_hbm.at[pl.ds(b1, CH)])
            return carry

        lax.fori_loop(0, per_w // (2 * CH), body, 0)

    return gk


def _neighbor_feats(K, MQ, g_ref, wk, wv):
    G = g_ref[...]
    gx = G[:, :, 0:128].reshape(K * MQ, 128)
    xfull = jnp.concatenate(
        [_unpack_lo(gx), _unpack_hi(gx)], axis=1).astype(BF16)
    k3 = jnp.dot(xfull, wk[...], preferred_element_type=F32)
    v3 = jnp.dot(xfull, wv[...], preferred_element_type=F32)
    nxyz = lax.bitcast_convert_type(G[:, :, 128:144], F32).reshape(K * MQ, 16)
    return k3, v3.reshape(K, MQ, 256), nxyz


def _attn_tail(K, MQ, a, v3, pe):
    a3 = a.reshape(K, MQ, 256) * (1.0 / 16.0)
    m = jnp.max(a3, axis=0, keepdims=True)
    e = jnp.exp(a3 - m)
    s = e / jnp.sum(e, axis=0, keepdims=True)
    vv = v3 + pe.reshape(K, MQ, 256)
    return jnp.sum(s * vv, axis=0)


def _attn_self_body(K, qx16_ref, g_ref, q_ref, pre_ref, wk, wv,
                    d1w, d1b, d2w, d2b, g1w, g1b, g2w, g2b, fc2w, fc2b,
                    out_ref):
    MQ = q_ref.shape[0]
    k3, v3, nxyz = _neighbor_feats(K, MQ, g_ref, wk, wv)
    qx = qx16_ref[...]
    pos = jnp.broadcast_to(qx[None], (K, MQ, 16)).reshape(K * MQ, 16) - nxyz
    pe = jnp.maximum(jnp.dot(pos, d1w[...], preferred_element_type=F32)
                     + d1b[...], 0.0)
    pe = jnp.dot(pe.astype(BF16), d2w[...], preferred_element_type=F32) + d2b[...]
    q = q_ref[...]
    qe = jnp.broadcast_to(q[None], (K, MQ, 256)).reshape(K * MQ, 256)
    t = (qe - k3 + pe).astype(BF16)
    a = jnp.maximum(jnp.dot(t, g1w[...], preferred_element_type=F32)
                    + g1b[...], 0.0)
    a = jnp.dot(a.astype(BF16), g2w[...], preferred_element_type=F32) + g2b[...]
    res = _attn_tail(K, MQ, a, v3, pe)
    out_ref[...] = (jnp.dot(res, fc2w[...], preferred_element_type=F32)
                    + fc2b[...] + pre_ref[...])


def _attn_cross_body(K, qx16_ref, g_ref, qf_ref, wk, wv,
                     wqx, wqf, d1w, d1b, d2w, d2b, g1w, g1b, g2w, g2b,
                     fc2w, fc2b, rbw1, rbb1, rbw2, rbb2,
                     out_ref, xyz_out):
    MQ = qf_ref.shape[0]
    k3, v3, nxyz = _neighbor_feats(K, MQ, g_ref, wk, wv)
    qx = qx16_ref[...]
    qf = qf_ref[...]
    q2 = (jnp.dot(qx, wqx[...], preferred_element_type=F32)
          + jnp.dot(qf, wqf[...], preferred_element_type=F32))
    pos = jnp.broadcast_to(qx[None], (K, MQ, 16)).reshape(K * MQ, 16) - nxyz
    pe = jnp.maximum(jnp.dot(pos, d1w[...], preferred_element_type=F32)
                     + d1b[...], 0.0)
    pe = jnp.dot(pe.astype(BF16), d2w[...], preferred_element_type=F32) + d2b[...]
    qe = jnp.broadcast_to(q2[None], (K, MQ, 256)).reshape(K * MQ, 256)
    t = (qe - k3 + pe).astype(BF16)
    a = jnp.maximum(jnp.dot(t, g1w[...], preferred_element_type=F32)
                    + g1b[...], 0.0)
    a = jnp.dot(a.astype(BF16), g2w[...], preferred_element_type=F32) + g2b[...]
    res = _attn_tail(K, MQ, a, v3, pe)
    qf2 = (jnp.dot(res, fc2w[...], preferred_element_type=F32)
           + fc2b[...] + qf)
    out_ref[...] = qf2
    h = jnp.maximum(jnp.dot(qf2, rbw1[...], preferred_element_type=F32)
                    + rbb1[...], 0.0)
    delta = jnp.dot(h, rbw2[...], preferred_element_type=F32) + rbb2[...]
    xyz_out[...] = delta + qx


def _attn_self(g3d, q_arr, pre, qx16f, wk, wv,
               d1w, d1b, d2w, d2b, g1w, g1b, g2w, g2b, fc2w, fc2b, MQ=256):
    R = q_arr.shape[0]
    K = g3d.shape[0]
    grid = (R // MQ,)
    return pl.pallas_call(
        functools.partial(_attn_self_body, K),
        grid=grid,
        in_specs=[
            pl.BlockSpec((MQ, 16), lambda i: (i, 0)),
            pl.BlockSpec((K, MQ, W_TAB), lambda i: (0, i, 0)),
            pl.BlockSpec((MQ, 256), lambda i: (i, 0)),
            pl.BlockSpec((MQ, 256), lambda i: (i, 0)),
            _wspec((256, 256)), _wspec((256, 256)),
            _wspec((16, 256)), _wspec((1, 256)),
            _wspec((256, 256)), _wspec((1, 256)),
            _wspec((256, 256)), _wspec((1, 256)),
            _wspec((256, 256)), _wspec((1, 256)),
            _wspec((256, 256)), _wspec((1, 256)),
        ],
        out_specs=pl.BlockSpec((MQ, 256), lambda i: (i, 0)),
        out_shape=jax.ShapeDtypeStruct((R, 256), F32),
    )(qx16f, g3d, q_arr, pre, wk, wv,
      d1w, d1b, d2w, d2b, g1w, g1b, g2w, g2b, fc2w, fc2b)


def _attn_cross(g3d, qf, qx16f, wk, wv, wqx, wqf,
                d1w, d1b, d2w, d2b, g1w, g1b, g2w, g2b, fc2w, fc2b,
                rbw1, rbb1, rbw2, rbb2, MQ=256):
    R = qf.shape[0]
    K = g3d.shape[0]
    grid = (R // MQ,)
    return pl.pallas_call(
        functools.partial(_attn_cross_body, K),
        grid=grid,
        in_specs=[
            pl.BlockSpec((MQ, 16), lambda i: (i, 0)),
            pl.BlockSpec((K, MQ, W_TAB), lambda i: (0, i, 0)),
            pl.BlockSpec((MQ, 256), lambda i: (i, 0)),
            _wspec((256, 256)), _wspec((256, 256)),
            _wspec((16, 256)), _wspec((256, 256)),
            _wspec((16, 256)), _wspec((1, 256)),
            _wspec((256, 256)), _wspec((1, 256)),
            _wspec((256, 256)), _wspec((1, 256)),
            _wspec((256, 256)), _wspec((1, 256)),
            _wspec((256, 256)), _wspec((1, 256)),
            _wspec((256, 256)), _wspec((1, 256)),
            _wspec((256, 16)), _wspec((1, 16)),
        ],
        out_specs=(
            pl.BlockSpec((MQ, 256), lambda i: (i, 0)),
            pl.BlockSpec((MQ, 16), lambda i: (i, 0)),
        ),
        out_shape=(
            jax.ShapeDtypeStruct((R, 256), F32),
            jax.ShapeDtypeStruct((R, 16), F32),
        ),
    )(qx16f, g3d, qf, wk, wv, wqx, wqf,
      d1w, d1b, d2w, d2b, g1w, g1b, g2w, g2b, fc2w, fc2b,
      rbw1, rbb1, rbw2, rbb2)


def kernel(query_xyz, query_feats, key_xyz, key_feats,
           sa_fc1_w, sa_fc2_w, sa_d1_w, sa_d2_w, sa_g1_w, sa_g2_w,
           sa_wq, sa_wk, sa_wv,
           ca_fc1_w, ca_fc2_w, ca_d1_w, ca_d2_w, ca_g1_w, ca_g2_w,
           ca_wq, ca_wk, ca_wv, rb_w1, rb_w2,
           sa_fc1_b, sa_fc2_b, sa_d1_b, sa_d2_b, sa_g1_b, sa_g2_b,
           ca_fc1_b, ca_fc2_b, ca_d1_b, ca_d2_b, ca_g1_b, ca_g2_b,
           rb_b1, rb_b2):
    B, NQ, _ = query_xyz.shape
    NK = key_xyz.shape[1]
    KQ, KC = 16, 32
    R = B * NQ

    def row(v):
        return v.reshape(1, -1)

    qx16 = jnp.pad(query_xyz, ((0, 0), (0, 0), (0, 13)))
    kx16 = jnp.pad(key_xyz, ((0, 0), (0, 0), (0, 13)))
    qxt16 = jnp.swapaxes(qx16, 1, 2)
    kxt16 = jnp.swapaxes(kx16, 1, 2)
    qx16f = qx16.reshape(R, 16)
    kx16f = kx16.reshape(B * NK, 16)

    sa_d1p = jnp.pad(sa_d1_w, ((0, 13), (0, 0)))
    ca_d1p = jnp.pad(ca_d1_w, ((0, 13), (0, 0)))
    wq_x = jnp.pad(ca_wq[:3], ((0, 13), (0, 0)))
    wq_f = ca_wq[3:]
    rb_w2p = jnp.pad(rb_w2, ((0, 0), (0, 13)))
    rb_b2p = jnp.pad(rb_b2, ((0, 13),)).reshape(1, 16)

    idx1 = _knn(qx16, qxt16, KQ, BQ=512)
    idx2 = _knn(qx16, kxt16, KC, BQ=512)
    idx1t = jnp.transpose(idx1).reshape(-1)
    idx2t = jnp.transpose(idx2).reshape(-1)

    qf_flat = query_feats.reshape(R, 256)
    kf_flat = key_feats.reshape(B * NK, 256)
    q_arr, tab1 = _proj_query(qf_flat, qx16f, sa_fc1_w, row(sa_fc1_b), sa_wq)
    tab2 = _proj_key(kf_flat, kx16f, ca_fc1_w, row(ca_fc1_b))

    g1 = _make_gather(KQ * R, W_TAB, GATHER_CHUNK)(tab1, idx1t)
    g2 = _make_gather(KC * R, W_TAB, GATHER_CHUNK)(tab2, idx2t)
    g1 = g1.reshape(KQ, R, W_TAB)
    g2 = g2.reshape(KC, R, W_TAB)

    qf1 = _attn_self(g1, q_arr, qf_flat, qx16f,
                     sa_wk.astype(BF16), sa_wv.astype(BF16),
                     sa_d1p, row(sa_d1_b), sa_d2_w.astype(BF16), row(sa_d2_b),
                     sa_g1_w.astype(BF16), row(sa_g1_b),
                     sa_g2_w.astype(BF16), row(sa_g2_b),
                     sa_fc2_w, row(sa_fc2_b))
    qf2, nx16 = _attn_cross(g2, qf1, qx16f,
                            ca_wk.astype(BF16), ca_wv.astype(BF16), wq_x, wq_f,
                            ca_d1p, row(ca_d1_b), ca_d2_w.astype(BF16), row(ca_d2_b),
                            ca_g1_w.astype(BF16), row(ca_g1_b),
                            ca_g2_w.astype(BF16), row(ca_g2_b),
                            ca_fc2_w, row(ca_fc2_b),
                            rb_w1, row(rb_b1), rb_w2p, rb_b2p)

    return qf2.reshape(B, NQ, 256), nx16.reshape(B, NQ, 16)[..., :3]


# speedup vs baseline: 1.0609x; 1.0609x over previous
import functools

import jax
import jax.numpy as jnp
import numpy as np
from jax import lax
from jax.experimental import pallas as pl
from jax.experimental.pallas import tpu as pltpu
from jax.experimental.pallas import tpu_sc as plsc

F32 = jnp.float32
BF16 = jnp.bfloat16
U32 = jnp.uint32
D = 256
W_TAB = 256
NUM_WORKERS = 32
GATHER_CHUNK = 128
HI_MASK = np.uint32(0xFFFF0000)


def _pack_bf16(x0, x1):
    a0 = lax.bitcast_convert_type(x0.astype(BF16).astype(F32), U32)
    a1 = lax.bitcast_convert_type(x1.astype(BF16).astype(F32), U32)
    return lax.bitcast_convert_type((a0 >> 16) | (a1 & HI_MASK), jnp.int32)


def _unpack_lo(gi):
    u = lax.bitcast_convert_type(gi, U32)
    return lax.bitcast_convert_type(u << 16, F32)


def _unpack_hi(gi):
    u = lax.bitcast_convert_type(gi, U32)
    return lax.bitcast_convert_type(u & HI_MASK, F32)


def _knn_body(K, N, qx_ref, kxt_ref, out_ref):
    qx = qx_ref[0]
    kxt = kxt_ref[0]
    b = pl.program_id(0)
    d = (jnp.sum(qx * qx, axis=1, keepdims=True)
         + jnp.sum(kxt * kxt, axis=0, keepdims=True)
         - 2.0 * jnp.dot(qx, kxt, preferred_element_type=F32))
    H = N // 2
    lo = d[:, :H]
    hi = d[:, H:]
    key = jnp.minimum(lo, hi)
    alt = jnp.maximum(lo, hi)
    iota = lax.broadcasted_iota(jnp.int32, key.shape, 1)
    oidx = jnp.where(hi < lo, iota + H, iota)
    cols = []
    for _ in range(K):
        m = jnp.min(key, axis=1, keepdims=True)
        oi = jnp.min(jnp.where(key <= m, oidx, 2 * N), axis=1, keepdims=True)
        cols.append(oi)
        cl = oi & (H - 1)
        pidx = jnp.where(oi >= H, oi - H, oi + H)
        eqc = iota == cl
        key = jnp.where(eqc, alt, key)
        alt = jnp.where(eqc, jnp.inf, alt)
        oidx = jnp.where(eqc, pidx, oidx)
    out_ref[...] = jnp.concatenate(cols, axis=1) + b * N


def _knn(qx16, kxt16, K, BQ):
    B, NQ, _ = qx16.shape
    N = kxt16.shape[2]
    grid = (B, NQ // BQ)
    return pl.pallas_call(
        functools.partial(_knn_body, K, N),
        grid=grid,
        in_specs=[
            pl.BlockSpec((1, BQ, 16), lambda b, q: (b, q, 0)),
            pl.BlockSpec((1, 16, N), lambda b, q: (b, 0, 0)),
        ],
        out_specs=pl.BlockSpec((BQ, K), lambda b, q: (b * (NQ // BQ) + q, 0)),
        out_shape=jax.ShapeDtypeStruct((B * NQ, K), jnp.int32),
    )(qx16, kxt16)


def _proj_q_body(qf_ref, qx16_ref, fc1w, fc1b, wq, q_out, tab_out):
    x = jnp.dot(qf_ref[...], fc1w[...], preferred_element_type=F32) + fc1b[...]
    q_out[...] = jnp.dot(x, wq[...], preferred_element_type=F32)
    qx = qx16_ref[...]
    tab_out[:, 0:128] = _pack_bf16(x[:, 0:128], x[:, 128:256])
    tab_out[:, 128:144] = lax.bitcast_convert_type(qx, jnp.int32)


def _proj_k_body(kf_ref, kx16_ref, fc1w, fc1b, tab_out):
    x = jnp.dot(kf_ref[...], fc1w[...], preferred_element_type=F32) + fc1b[...]
    kx = kx16_ref[...]
    tab_out[:, 0:128] = _pack_bf16(x[:, 0:128], x[:, 128:256])
    tab_out[:, 128:144] = lax.bitcast_convert_type(kx, jnp.int32)


def _wspec(shape):
    return pl.BlockSpec(shape, lambda i: (0,) * len(shape))


def _proj_query(qf, qx16, fc1w, fc1b, wq, RP=512):
    R = qf.shape[0]
    grid = (R // RP,)
    return pl.pallas_call(
        _proj_q_body,
        grid=grid,
        in_specs=[
            pl.BlockSpec((RP, 256), lambda i: (i, 0)),
            pl.BlockSpec((RP, 16), lambda i: (i, 0)),
            _wspec((256, 256)), _wspec((1, 256)),
            _wspec((256, 256)),
        ],
        out_specs=(
            pl.BlockSpec((RP, 256), lambda i: (i, 0)),
            pl.BlockSpec((RP, W_TAB), lambda i: (i, 0)),
        ),
        out_shape=(
            jax.ShapeDtypeStruct((R, 256), F32),
            jax.ShapeDtypeStruct((R, W_TAB), jnp.int32),
        ),
    )(qf, qx16, fc1w, fc1b, wq)


def _proj_key(kf, kx16, fc1w, fc1b, RP=512):
    R = kf.shape[0]
    grid = (R // RP,)
    return pl.pallas_call(
        _proj_k_body,
        grid=grid,
        in_specs=[
            pl.BlockSpec((RP, 256), lambda i: (i, 0)),
            pl.BlockSpec((RP, 16), lambda i: (i, 0)),
            _wspec((256, 256)), _wspec((1, 256)),
        ],
        out_specs=pl.BlockSpec((RP, W_TAB), lambda i: (i, 0)),
        out_shape=jax.ShapeDtypeStruct((R, W_TAB), jnp.int32),
    )(kf, kx16, fc1w, fc1b)


def _make_gather(R, W, CH):
    mesh = plsc.VectorSubcoreMesh(core_axis_name="c", subcore_axis_name="s")
    per_w = R // NUM_WORKERS

    @functools.partial(
        pl.kernel, mesh=mesh,
        out_type=jax.ShapeDtypeStruct((R, W), jnp.int32),
        scratch_types=[
            pltpu.VMEM((CH,), jnp.int32),
            pltpu.VMEM((CH,), jnp.int32),
            pltpu.VMEM((CH, W), jnp.int32),
            pltpu.VMEM((CH, W), jnp.int32),
            pltpu.SemaphoreType.DMA,
            pltpu.SemaphoreType.DMA,
        ],
    )
    def gk(tab_hbm, idx_hbm, out_hbm, idx_v0, idx_v1, rows_v0, rows_v1,
           sem0, sem1):
        wid = lax.axis_index("s") * 2 + lax.axis_index("c")
        base = wid * per_w

        def body(j, carry):
            b0 = base + (2 * j) * CH
            b1 = b0 + CH
            pltpu.sync_copy(idx_hbm.at[pl.ds(b0, CH)], idx_v0)
            g0 = pltpu.async_copy(tab_hbm.at[idx_v0], rows_v0, sem0)
            pltpu.sync_copy(idx_hbm.at[pl.ds(b1, CH)], idx_v1)
            g1 = pltpu.async_copy(tab_hbm.at[idx_v1], rows_v1, sem1)
            g0.wait()
            pltpu.sync_copy(rows_v0, out_hbm.at[pl.ds(b0, CH)])
            g1.wait()
            pltpu.sync_copy(rows_v1, out_hbm.at[pl.ds(b1, CH)])
            return carry

        lax.fori_loop(0, per_w // (2 * CH), body, 0)

    return gk


def _neighbor_feats(K, MQ, g_ref, wk, wv):
    G = g_ref[...]
    gx = G[:, :, 0:128].reshape(K * MQ, 128)
    xfull = jnp.concatenate(
        [_unpack_lo(gx), _unpack_hi(gx)], axis=1).astype(BF16)
    k3 = jnp.dot(xfull, wk[...], preferred_element_type=F32)
    v3 = jnp.dot(xfull, wv[...], preferred_element_type=F32)
    nxyz = lax.bitcast_convert_type(G[:, :, 128:144], F32).reshape(K * MQ, 16)
    return k3, v3.reshape(K, MQ, 256), nxyz


def _attn_tail(K, MQ, a, v3, pe):
    a3 = a.reshape(K, MQ, 256) * (1.0 / 16.0)
    m = jnp.max(a3, axis=0, keepdims=True)
    e = jnp.exp(a3 - m)
    s = e / jnp.sum(e, axis=0, keepdims=True)
    vv = v3 + pe.reshape(K, MQ, 256)
    return jnp.sum(s * vv, axis=0)


def _attn_self_body(K, qx16_ref, g_ref, q_ref, pre_ref, wk, wv,
                    d1w, d1b, d2w, d2b, g1w, g1b, g2w, g2b, fc2w, fc2b,
                    out_ref):
    MQ = q_ref.shape[0]
    k3, v3, nxyz = _neighbor_feats(K, MQ, g_ref, wk, wv)
    qx = qx16_ref[...]
    pos = jnp.broadcast_to(qx[None], (K, MQ, 16)).reshape(K * MQ, 16) - nxyz
    pe = jnp.maximum(jnp.dot(pos, d1w[...], preferred_element_type=F32)
                     + d1b[...], 0.0)
    pe = jnp.dot(pe.astype(BF16), d2w[...], preferred_element_type=F32) + d2b[...]
    q = q_ref[...]
    qe = jnp.broadcast_to(q[None], (K, MQ, 256)).reshape(K * MQ, 256)
    t = (qe - k3 + pe).astype(BF16)
    a = jnp.maximum(jnp.dot(t, g1w[...], preferred_element_type=F32)
                    + g1b[...], 0.0)
    a = jnp.dot(a.astype(BF16), g2w[...], preferred_element_type=F32) + g2b[...]
    res = _attn_tail(K, MQ, a, v3, pe)
    out_ref[...] = (jnp.dot(res, fc2w[...], preferred_element_type=F32)
                    + fc2b[...] + pre_ref[...])


def _attn_cross_body(K, qx16_ref, g_ref, qf_ref, wk, wv,
                     wqx, wqf, d1w, d1b, d2w, d2b, g1w, g1b, g2w, g2b,
                     fc2w, fc2b, rbw1, rbb1, rbw2, rbb2,
                     out_ref, xyz_out):
    MQ = qf_ref.shape[0]
    k3, v3, nxyz = _neighbor_feats(K, MQ, g_ref, wk, wv)
    qx = qx16_ref[...]
    qf = qf_ref[...]
    q2 = (jnp.dot(qx, wqx[...], preferred_element_type=F32)
          + jnp.dot(qf, wqf[...], preferred_element_type=F32))
    pos = jnp.broadcast_to(qx[None], (K, MQ, 16)).reshape(K * MQ, 16) - nxyz
    pe = jnp.maximum(jnp.dot(pos, d1w[...], preferred_element_type=F32)
                     + d1b[...], 0.0)
    pe = jnp.dot(pe.astype(BF16), d2w[...], preferred_element_type=F32) + d2b[...]
    qe = jnp.broadcast_to(q2[None], (K, MQ, 256)).reshape(K * MQ, 256)
    t = (qe - k3 + pe).astype(BF16)
    a = jnp.maximum(jnp.dot(t, g1w[...], preferred_element_type=F32)
                    + g1b[...], 0.0)
    a = jnp.dot(a.astype(BF16), g2w[...], preferred_element_type=F32) + g2b[...]
    res = _attn_tail(K, MQ, a, v3, pe)
    qf2 = (jnp.dot(res, fc2w[...], preferred_element_type=F32)
           + fc2b[...] + qf)
    out_ref[...] = qf2
    h = jnp.maximum(jnp.dot(qf2, rbw1[...], preferred_element_type=F32)
                    + rbb1[...], 0.0)
    delta = jnp.dot(h, rbw2[...], preferred_element_type=F32) + rbb2[...]
    xyz_out[...] = delta + qx


def _attn_self(g3d, q_arr, pre, qx16f, wk, wv,
               d1w, d1b, d2w, d2b, g1w, g1b, g2w, g2b, fc2w, fc2b, MQ=256):
    R = q_arr.shape[0]
    K = g3d.shape[0]
    grid = (R // MQ,)
    return pl.pallas_call(
        functools.partial(_attn_self_body, K),
        grid=grid,
        in_specs=[
            pl.BlockSpec((MQ, 16), lambda i: (i, 0)),
            pl.BlockSpec((K, MQ, W_TAB), lambda i: (0, i, 0)),
            pl.BlockSpec((MQ, 256), lambda i: (i, 0)),
            pl.BlockSpec((MQ, 256), lambda i: (i, 0)),
            _wspec((256, 256)), _wspec((256, 256)),
            _wspec((16, 256)), _wspec((1, 256)),
            _wspec((256, 256)), _wspec((1, 256)),
            _wspec((256, 256)), _wspec((1, 256)),
            _wspec((256, 256)), _wspec((1, 256)),
            _wspec((256, 256)), _wspec((1, 256)),
        ],
        out_specs=pl.BlockSpec((MQ, 256), lambda i: (i, 0)),
        out_shape=jax.ShapeDtypeStruct((R, 256), F32),
    )(qx16f, g3d, q_arr, pre, wk, wv,
      d1w, d1b, d2w, d2b, g1w, g1b, g2w, g2b, fc2w, fc2b)


def _attn_cross(g3d, qf, qx16f, wk, wv, wqx, wqf,
                d1w, d1b, d2w, d2b, g1w, g1b, g2w, g2b, fc2w, fc2b,
                rbw1, rbb1, rbw2, rbb2, MQ=256):
    R = qf.shape[0]
    K = g3d.shape[0]
    grid = (R // MQ,)
    return pl.pallas_call(
        functools.partial(_attn_cross_body, K),
        grid=grid,
        in_specs=[
            pl.BlockSpec((MQ, 16), lambda i: (i, 0)),
            pl.BlockSpec((K, MQ, W_TAB), lambda i: (0, i, 0)),
            pl.BlockSpec((MQ, 256), lambda i: (i, 0)),
            _wspec((256, 256)), _wspec((256, 256)),
            _wspec((16, 256)), _wspec((256, 256)),
            _wspec((16, 256)), _wspec((1, 256)),
            _wspec((256, 256)), _wspec((1, 256)),
            _wspec((256, 256)), _wspec((1, 256)),
            _wspec((256, 256)), _wspec((1, 256)),
            _wspec((256, 256)), _wspec((1, 256)),
            _wspec((256, 256)), _wspec((1, 256)),
            _wspec((256, 16)), _wspec((1, 16)),
        ],
        out_specs=(
            pl.BlockSpec((MQ, 256), lambda i: (i, 0)),
            pl.BlockSpec((MQ, 16), lambda i: (i, 0)),
        ),
        out_shape=(
            jax.ShapeDtypeStruct((R, 256), F32),
            jax.ShapeDtypeStruct((R, 16), F32),
        ),
    )(qx16f, g3d, qf, wk, wv, wqx, wqf,
      d1w, d1b, d2w, d2b, g1w, g1b, g2w, g2b, fc2w, fc2b,
      rbw1, rbb1, rbw2, rbb2)


def kernel(query_xyz, query_feats, key_xyz, key_feats,
           sa_fc1_w, sa_fc2_w, sa_d1_w, sa_d2_w, sa_g1_w, sa_g2_w,
           sa_wq, sa_wk, sa_wv,
           ca_fc1_w, ca_fc2_w, ca_d1_w, ca_d2_w, ca_g1_w, ca_g2_w,
           ca_wq, ca_wk, ca_wv, rb_w1, rb_w2,
           sa_fc1_b, sa_fc2_b, sa_d1_b, sa_d2_b, sa_g1_b, sa_g2_b,
           ca_fc1_b, ca_fc2_b, ca_d1_b, ca_d2_b, ca_g1_b, ca_g2_b,
           rb_b1, rb_b2):
    B, NQ, _ = query_xyz.shape
    NK = key_xyz.shape[1]
    KQ, KC = 16, 32
    R = B * NQ

    def row(v):
        return v.reshape(1, -1)

    qx16 = jnp.pad(query_xyz, ((0, 0), (0, 0), (0, 13)))
    kx16 = jnp.pad(key_xyz, ((0, 0), (0, 0), (0, 13)))
    qxt16 = jnp.swapaxes(qx16, 1, 2)
    kxt16 = jnp.swapaxes(kx16, 1, 2)
    qx16f = qx16.reshape(R, 16)
    kx16f = kx16.reshape(B * NK, 16)

    sa_d1p = jnp.pad(sa_d1_w, ((0, 13), (0, 0)))
    ca_d1p = jnp.pad(ca_d1_w, ((0, 13), (0, 0)))
    wq_x = jnp.pad(ca_wq[:3], ((0, 13), (0, 0)))
    wq_f = ca_wq[3:]
    rb_w2p = jnp.pad(rb_w2, ((0, 0), (0, 13)))
    rb_b2p = jnp.pad(rb_b2, ((0, 13),)).reshape(1, 16)

    idx1 = _knn(qx16, qxt16, KQ, BQ=256)
    idx2 = _knn(qx16, kxt16, KC, BQ=256)
    idx1t = jnp.transpose(idx1).reshape(-1)
    idx2t = jnp.transpose(idx2).reshape(-1)

    qf_flat = query_feats.reshape(R, 256)
    kf_flat = key_feats.reshape(B * NK, 256)
    q_arr, tab1 = _proj_query(qf_flat, qx16f, sa_fc1_w, row(sa_fc1_b), sa_wq)
    tab2 = _proj_key(kf_flat, kx16f, ca_fc1_w, row(ca_fc1_b))

    g1 = _make_gather(KQ * R, W_TAB, GATHER_CHUNK)(tab1, idx1t)
    g2 = _make_gather(KC * R, W_TAB, GATHER_CHUNK)(tab2, idx2t)
    g1 = g1.reshape(KQ, R, W_TAB)
    g2 = g2.reshape(KC, R, W_TAB)

    qf1 = _attn_self(g1, q_arr, qf_flat, qx16f,
                     sa_wk.astype(BF16), sa_wv.astype(BF16),
                     sa_d1p, row(sa_d1_b), sa_d2_w.astype(BF16), row(sa_d2_b),
                     sa_g1_w.astype(BF16), row(sa_g1_b),
                     sa_g2_w.astype(BF16), row(sa_g2_b),
                     sa_fc2_w, row(sa_fc2_b))
    qf2, nx16 = _attn_cross(g2, qf1, qx16f,
                            ca_wk.astype(BF16), ca_wv.astype(BF16), wq_x, wq_f,
                            ca_d1p, row(ca_d1_b), ca_d2_w.astype(BF16), row(ca_d2_b),
                            ca_g1_w.astype(BF16), row(ca_g1_b),
                            ca_g2_w.astype(BF16), row(ca_g2_b),
                            ca_fc2_w, row(ca_fc2_b),
                            rb_w1, row(rb_b1), rb_w2p, rb_b2p)

    return qf2.reshape(B, NQ, 256), nx16.reshape(B, NQ, 16)[..., :3]
